# baseline (speedup 1.0000x reference)
import functools

import jax
import jax.numpy as jnp
from jax import lax
from jax.experimental import pallas as pl
from jax.experimental.pallas import tpu as pltpu
from jax.experimental.pallas import tpu_sc as plsc

N = 10000
E = 320000
F = 128

NC = 2
NS = 16
G = 128
NG = 2560
GPT = NG // (NC * NS)
N_PAD = 10240
RPT = N_PAD // NS
RCH = 64
CH = 8
NCH = RPT // RCH
BLK = 1024
TRASH = N

_MESH = plsc.VectorSubcoreMesh(
    core_axis_name="c", subcore_axis_name="s", num_cores=NC, num_subcores=NS
)


@functools.partial(
    pl.kernel,
    out_type=jax.ShapeDtypeStruct((NC, N_PAD), jnp.float32),
    mesh=_MESH,
    scratch_types=[
        pltpu.VMEM((GPT, G), jnp.int32),
        pltpu.VMEM((GPT, G), jnp.int32),
        pltpu.VMEM((G,), jnp.float32),
        pltpu.VMEM((RPT,), jnp.float32),
        pltpu.VMEM_SHARED((N_PAD,), jnp.float32),
        pltpu.SemaphoreType.DMA,
    ],
)
def _sc_hist(row_hbm, col_hbm, out_hbm, rowb, colb, onesb, stage, deg_sh, sem):
    c = lax.axis_index("c")
    s = lax.axis_index("s")

    def _z(i, _):
        stage[pl.ds(i * 16, 16)] = jnp.zeros((16,), jnp.float32)
        return 0

    lax.fori_loop(0, RPT // 16, _z, 0)
    pltpu.sync_copy(stage, deg_sh.at[pl.ds(s * RPT, RPT)])
    plsc.subcore_barrier()

    gbase = (c * NS + s) * GPT
    pltpu.sync_copy(row_hbm.at[pl.ds(gbase, GPT)], rowb)
    pltpu.sync_copy(col_hbm.at[pl.ds(gbase, GPT)], colb)

    def _group(g, _):
        def _mask(i, _):
            r = rowb[g, pl.ds(i * 16, 16)]
            cc = colb[g, pl.ds(i * 16, 16)]
            onesb[pl.ds(i * 16, 16)] = jnp.where(r != cc, 1.0, 0.0)
            return 0

        lax.fori_loop(0, G // 16, _mask, 0)
        pltpu.sync_copy(onesb, deg_sh.at[rowb.at[g]], add=True)
        return 0

    lax.fori_loop(0, GPT, _group, 0)
    plsc.subcore_barrier()

    pltpu.sync_copy(deg_sh.at[pl.ds(s * RPT, RPT)],
                    out_hbm.at[c, pl.ds(s * RPT, RPT)])


@functools.partial(
    pl.kernel,
    out_type=jax.ShapeDtypeStruct((NC, N_PAD, F), jnp.float32),
    mesh=_MESH,
    scratch_types=[
        pltpu.VMEM((CH, G), jnp.int32),
        pltpu.VMEM((CH, G), jnp.int32),
        pltpu.VMEM((G, F), jnp.float32),
        pltpu.VMEM((G, F), jnp.float32),
        pltpu.VMEM((RCH, F), jnp.float32),
        pltpu.VMEM_SHARED((N_PAD, F), jnp.float32),
        pltpu.SemaphoreType.DMA,
        pltpu.SemaphoreType.DMA,
        pltpu.SemaphoreType.DMA,
        pltpu.SemaphoreType.DMA,
    ],
)
def _sc_scatter(row_hbm, col_hbm, v_hbm, out_hbm, rowb, colb, rows0, rows1,
                stage, acc_sh, sem0, sem1, ssem0, ssem1):
    c = lax.axis_index("c")
    s = lax.axis_index("s")

    def _z(i, _):
        for j in range(F // 16):
            stage[i, pl.ds(j * 16, 16)] = jnp.zeros((16,), jnp.float32)
        return 0

    lax.fori_loop(0, RCH, _z, 0)
    for k in range(NCH):
        pltpu.sync_copy(stage, acc_sh.at[pl.ds(s * RPT + k * RCH, RCH), :])
    plsc.subcore_barrier()

    gbase = (c * NS + s) * GPT
    lane = lax.broadcasted_iota(jnp.int32, (16,), 0)
    bufs = (rows0, rows1)
    sems = (sem0, sem1)
    ssems = (ssem0, ssem1)

    def _chunk(ch, _):
        base = gbase + ch * CH
        pltpu.sync_copy(row_hbm.at[pl.ds(base, CH)], rowb)
        pltpu.sync_copy(col_hbm.at[pl.ds(base, CH)], colb)

        def _mask(t, _):
            g = t // 8
            i = t % 8
            r = rowb[g, pl.ds(i * 16, 16)]
            cc = colb[g, pl.ds(i * 16, 16)]
            trash = TRASH + ((t + ch) % 15) * 16 + lane
            colb[g, pl.ds(i * 16, 16)] = jnp.where(r == cc, trash, cc)
            return 0

        lax.fori_loop(0, CH * 8, _mask, 0)

        desc = [None] * CH
        sdesc = [None] * CH
        desc[0] = pltpu.async_copy(v_hbm.at[rowb.at[0]], bufs[0], sems[0])
        for g in range(CH):
            p = g % 2
            desc[g].wait()
            sdesc[g] = pltpu.async_copy(
                bufs[p], acc_sh.at[colb.at[g]], ssems[p], add=True)
            if g + 1 < CH:
                if g >= 1:
                    sdesc[g - 1].wait()
                desc[g + 1] = pltpu.async_copy(
                    v_hbm.at[rowb.at[g + 1]], bufs[1 - p], sems[1 - p])
        sdesc[CH - 2].wait()
        sdesc[CH - 1].wait()
        return 0

    lax.fori_loop(0, GPT // CH, _chunk, 0)
    plsc.subcore_barrier()

    pltpu.sync_copy(acc_sh.at[pl.ds(s * RPT, RPT), :],
                    out_hbm.at[c, pl.ds(s * RPT, RPT), :])


def _dinv_sublane(deg_ref, i):
    ri = lax.broadcasted_iota(jnp.int32, (128, 128), 0)
    ci = lax.broadcasted_iota(jnp.int32, (128, 128), 1)
    pieces = []
    for j in range(BLK // 128):
        dsl = pl.ds(i * BLK + j * 128, 128)
        d = deg_ref[0:1, dsl] + deg_ref[1:2, dsl]
        dinv = jnp.where(d > 0, lax.rsqrt(d), 0.0)
        pieces.append(
            jnp.sum(jnp.where(ri == ci, dinv, 0.0), axis=1, keepdims=True)
        )
    return jnp.concatenate(pieces, axis=0)


def _tc_mm_body(x_ref, w0_ref, w1_ref, b_ref, y0_ref, y1_ref):
    xb = x_ref[...]
    y0_ref[...] = (
        jnp.dot(xb, w0_ref[...], preferred_element_type=jnp.float32)
        + b_ref[...]
    )
    y1_ref[...] = jnp.dot(xb, w1_ref[...], preferred_element_type=jnp.float32)


def _tc_vscale_body(deg_ref, y1_ref, v_ref):
    i = pl.program_id(0)
    v_ref[...] = _dinv_sublane(deg_ref, i) * y1_ref[...]


def _tc_final_body(deg_ref, y0_ref, a_ref, o_ref):
    i = pl.program_id(0)
    dinv_sub = _dinv_sublane(deg_ref, i)
    acc = a_ref[0] + a_ref[1]
    o_ref[...] = jnp.maximum(y0_ref[...] - dinv_sub * acc, 0.0)


_tc_mm = pl.pallas_call(
    _tc_mm_body,
    grid=(N_PAD // BLK,),
    in_specs=[
        pl.BlockSpec((BLK, F), lambda i: (i, 0)),
        pl.BlockSpec((F, F), lambda i: (0, 0)),
        pl.BlockSpec((F, F), lambda i: (0, 0)),
        pl.BlockSpec((1, F), lambda i: (0, 0)),
    ],
    out_specs=[
        pl.BlockSpec((BLK, F), lambda i: (i, 0)),
        pl.BlockSpec((BLK, F), lambda i: (i, 0)),
    ],
    out_shape=[
        jax.ShapeDtypeStruct((N_PAD, F), jnp.float32),
        jax.ShapeDtypeStruct((N_PAD, F), jnp.float32),
    ],
)

_tc_vscale = pl.pallas_call(
    _tc_vscale_body,
    grid=(N_PAD // BLK,),
    in_specs=[
        pl.BlockSpec((NC, N_PAD), lambda i: (0, 0)),
        pl.BlockSpec((BLK, F), lambda i: (i, 0)),
    ],
    out_specs=pl.BlockSpec((BLK, F), lambda i: (i, 0)),
    out_shape=jax.ShapeDtypeStruct((N_PAD, F), jnp.float32),
)

_tc_final = pl.pallas_call(
    _tc_final_body,
    grid=(N_PAD // BLK,),
    in_specs=[
        pl.BlockSpec((NC, N_PAD), lambda i: (0, 0)),
        pl.BlockSpec((BLK, F), lambda i: (i, 0)),
        pl.BlockSpec((NC, BLK, F), lambda i: (0, i, 0)),
    ],
    out_specs=pl.BlockSpec((BLK, F), lambda i: (i, 0)),
    out_shape=jax.ShapeDtypeStruct((N_PAD, F), jnp.float32),
)


@jax.jit
def kernel(x, adj, W0, W1, b):
    pad = NG * G - E
    pidx = TRASH + jnp.arange(pad, dtype=jnp.int32) % (N_PAD - N)
    rowp = jnp.concatenate([adj[0], pidx]).reshape(NG, G)
    colp = jnp.concatenate([adj[1], pidx]).reshape(NG, G)
    x_pad = jnp.pad(x, ((0, N_PAD - N), (0, 0)))

    y0, y1 = _tc_mm(x_pad, W0, W1, b.reshape(1, F))
    deg = _sc_hist(rowp, colp)
    v = _tc_vscale(deg, y1)
    acc = _sc_scatter(rowp, colp, v)
    out = _tc_final(deg, y0, acc)
    return out[:N]


# speedup vs baseline: 1.1002x; 1.1002x over previous
import functools

import jax
import jax.numpy as jnp
from jax import lax
from jax.experimental import pallas as pl
from jax.experimental.pallas import tpu as pltpu
from jax.experimental.pallas import tpu_sc as plsc

N = 10000
E = 320000
F = 128

NC = 2
NS = 16
G = 128
NG = 2560
GPT = NG // (NC * NS)
N_PAD = 10240
RPT = N_PAD // NS
RCH = 64
CH = 8
NCH = RPT // RCH
BLK = 1024
TRASH = N

_MESH = plsc.VectorSubcoreMesh(
    core_axis_name="c", subcore_axis_name="s", num_cores=NC, num_subcores=NS
)


@functools.partial(
    pl.kernel,
    out_type=jax.ShapeDtypeStruct((NC, N_PAD), jnp.float32),
    mesh=_MESH,
    scratch_types=[
        pltpu.VMEM((GPT, G), jnp.int32),
        pltpu.VMEM((GPT, G), jnp.int32),
        pltpu.VMEM((G,), jnp.float32),
        pltpu.VMEM((RPT,), jnp.float32),
        pltpu.VMEM_SHARED((N_PAD,), jnp.float32),
        pltpu.SemaphoreType.DMA,
    ],
)
def _sc_hist(row_hbm, col_hbm, out_hbm, rowb, colb, onesb, stage, deg_sh, sem):
    c = lax.axis_index("c")
    s = lax.axis_index("s")

    def _z(i, _):
        stage[pl.ds(i * 16, 16)] = jnp.zeros((16,), jnp.float32)
        return 0

    lax.fori_loop(0, RPT // 16, _z, 0)
    pltpu.sync_copy(stage, deg_sh.at[pl.ds(s * RPT, RPT)])
    plsc.subcore_barrier()

    gbase = (c * NS + s) * GPT
    pltpu.sync_copy(row_hbm.at[pl.ds(gbase, GPT)], rowb)
    pltpu.sync_copy(col_hbm.at[pl.ds(gbase, GPT)], colb)

    def _group(g, _):
        def _mask(i, _):
            r = rowb[g, pl.ds(i * 16, 16)]
            cc = colb[g, pl.ds(i * 16, 16)]
            onesb[pl.ds(i * 16, 16)] = jnp.where(r != cc, 1.0, 0.0)
            return 0

        lax.fori_loop(0, G // 16, _mask, 0)
        pltpu.sync_copy(onesb, deg_sh.at[rowb.at[g]], add=True)
        return 0

    lax.fori_loop(0, GPT, _group, 0)
    plsc.subcore_barrier()

    pltpu.sync_copy(deg_sh.at[pl.ds(s * RPT, RPT)],
                    out_hbm.at[c, pl.ds(s * RPT, RPT)])


@functools.partial(
    pl.kernel,
    out_type=jax.ShapeDtypeStruct((NC, N_PAD, F), jnp.float32),
    mesh=_MESH,
    scratch_types=[
        pltpu.VMEM((CH, G), jnp.int32),
        pltpu.VMEM((CH, G), jnp.int32),
        pltpu.VMEM((G, F), jnp.float32),
        pltpu.VMEM((G, F), jnp.float32),
        pltpu.VMEM((RCH, F), jnp.float32),
        pltpu.VMEM_SHARED((N_PAD, F), jnp.float32),
        pltpu.SemaphoreType.DMA,
        pltpu.SemaphoreType.DMA,
    ],
)
def _sc_scatter(row_hbm, col_hbm, v_hbm, out_hbm, rowb, colb, rows0, rows1,
                stage, acc_sh, sem0, sem1):
    c = lax.axis_index("c")
    s = lax.axis_index("s")

    def _z(i, _):
        for j in range(F // 16):
            stage[i, pl.ds(j * 16, 16)] = jnp.zeros((16,), jnp.float32)
        return 0

    lax.fori_loop(0, RCH, _z, 0)
    for k in range(NCH):
        pltpu.sync_copy(stage, acc_sh.at[pl.ds(s * RPT + k * RCH, RCH), :])
    plsc.subcore_barrier()

    gbase = (c * NS + s) * GPT
    lane = lax.broadcasted_iota(jnp.int32, (16,), 0)
    bufs = (rows0, rows1)
    sems = (sem0, sem1)

    def _chunk(ch, _):
        base = gbase + ch * CH
        pltpu.sync_copy(row_hbm.at[pl.ds(base, CH)], rowb)
        pltpu.sync_copy(col_hbm.at[pl.ds(base, CH)], colb)

        def _mask(t, _):
            g = t // 8
            i = t % 8
            r = rowb[g, pl.ds(i * 16, 16)]
            cc = colb[g, pl.ds(i * 16, 16)]
            trash = TRASH + ((t + ch) % 15) * 16 + lane
            colb[g, pl.ds(i * 16, 16)] = jnp.where(r == cc, trash, cc)
            return 0

        lax.fori_loop(0, CH * 8, _mask, 0)

        desc = [None] * CH
        desc[0] = pltpu.async_copy(v_hbm.at[rowb.at[0]], bufs[0], sems[0])
        for g in range(CH):
            if g + 1 < CH:
                desc[g + 1] = pltpu.async_copy(
                    v_hbm.at[rowb.at[g + 1]], bufs[(g + 1) % 2],
                    sems[(g + 1) % 2])
            desc[g].wait()
            pltpu.sync_copy(bufs[g % 2], acc_sh.at[colb.at[g]], add=True)
        return 0

    lax.fori_loop(0, GPT // CH, _chunk, 0)
    plsc.subcore_barrier()

    pltpu.sync_copy(acc_sh.at[pl.ds(s * RPT, RPT), :],
                    out_hbm.at[c, pl.ds(s * RPT, RPT), :])


def _dinv_sublane(deg_ref, i):
    ri = lax.broadcasted_iota(jnp.int32, (128, 128), 0)
    ci = lax.broadcasted_iota(jnp.int32, (128, 128), 1)
    pieces = []
    for j in range(BLK // 128):
        dsl = pl.ds(i * BLK + j * 128, 128)
        d = deg_ref[0:1, dsl] + deg_ref[1:2, dsl]
        dinv = jnp.where(d > 0, lax.rsqrt(d), 0.0)
        pieces.append(
            jnp.sum(jnp.where(ri == ci, dinv, 0.0), axis=1, keepdims=True)
        )
    return jnp.concatenate(pieces, axis=0)


def _tc_mm_body(x_ref, w0_ref, w1_ref, b_ref, y0_ref, y1_ref):
    xb = x_ref[...]
    y0_ref[...] = (
        jnp.dot(xb, w0_ref[...], preferred_element_type=jnp.float32)
        + b_ref[...]
    )
    y1_ref[...] = jnp.dot(xb, w1_ref[...], preferred_element_type=jnp.float32)


def _tc_vscale_body(deg_ref, y1_ref, v_ref):
    i = pl.program_id(0)
    v_ref[...] = _dinv_sublane(deg_ref, i) * y1_ref[...]


def _tc_final_body(deg_ref, y0_ref, a_ref, o_ref):
    i = pl.program_id(0)
    dinv_sub = _dinv_sublane(deg_ref, i)
    acc = a_ref[0] + a_ref[1]
    o_ref[...] = jnp.maximum(y0_ref[...] - dinv_sub * acc, 0.0)


_tc_mm = pl.pallas_call(
    _tc_mm_body,
    grid=(N_PAD // BLK,),
    in_specs=[
        pl.BlockSpec((BLK, F), lambda i: (i, 0)),
        pl.BlockSpec((F, F), lambda i: (0, 0)),
        pl.BlockSpec((F, F), lambda i: (0, 0)),
        pl.BlockSpec((1, F), lambda i: (0, 0)),
    ],
    out_specs=[
        pl.BlockSpec((BLK, F), lambda i: (i, 0)),
        pl.BlockSpec((BLK, F), lambda i: (i, 0)),
    ],
    out_shape=[
        jax.ShapeDtypeStruct((N, F), jnp.float32),
        jax.ShapeDtypeStruct((N, F), jnp.float32),
    ],
)

_tc_vscale = pl.pallas_call(
    _tc_vscale_body,
    grid=(N_PAD // BLK,),
    in_specs=[
        pl.BlockSpec((NC, N_PAD), lambda i: (0, 0)),
        pl.BlockSpec((BLK, F), lambda i: (i, 0)),
    ],
    out_specs=pl.BlockSpec((BLK, F), lambda i: (i, 0)),
    out_shape=jax.ShapeDtypeStruct((N_PAD, F), jnp.float32),
)

_tc_final = pl.pallas_call(
    _tc_final_body,
    grid=(N_PAD // BLK,),
    in_specs=[
        pl.BlockSpec((NC, N_PAD), lambda i: (0, 0)),
        pl.BlockSpec((BLK, F), lambda i: (i, 0)),
        pl.BlockSpec((NC, BLK, F), lambda i: (0, i, 0)),
    ],
    out_specs=pl.BlockSpec((BLK, F), lambda i: (i, 0)),
    out_shape=jax.ShapeDtypeStruct((N, F), jnp.float32),
)


@jax.jit
def kernel(x, adj, W0, W1, b):
    pad = NG * G - E
    pidx = TRASH + jnp.arange(pad, dtype=jnp.int32) % (N_PAD - N)
    rowp = jnp.concatenate([adj[0], pidx]).reshape(NG, G)
    colp = jnp.concatenate([adj[1], pidx]).reshape(NG, G)
    y0, y1 = _tc_mm(x, W0, W1, b.reshape(1, F))
    deg = _sc_hist(rowp, colp)
    v = _tc_vscale(deg, y1)
    acc = _sc_scatter(rowp, colp, v)
    return _tc_final(deg, y0, acc)


# speedup vs baseline: 1.1722x; 1.0654x over previous
import functools

import jax
import jax.numpy as jnp
from jax import lax
from jax.experimental import pallas as pl
from jax.experimental.pallas import tpu as pltpu
from jax.experimental.pallas import tpu_sc as plsc

N = 10000
E = 320000
F = 128

NC = 2
NS = 16
G = 128
CH = 8
ECH = CH * G
NBIG = 24
TAILB = 312 * ECH
N_PAD = 10240
RPT = N_PAD // NS
RCH = 64
NCH = RPT // RCH
BLK = 1024
TRASH = N

_MESH = plsc.VectorSubcoreMesh(
    core_axis_name="c", subcore_axis_name="s", num_cores=NC, num_subcores=NS
)


def _edge_base(wid):
    big = jnp.minimum(wid, NBIG)
    return big * (10 * ECH) + jnp.maximum(wid - NBIG, 0) * (9 * ECH)


@functools.partial(
    pl.kernel,
    out_type=jax.ShapeDtypeStruct((NC, N_PAD), jnp.float32),
    mesh=_MESH,
    scratch_types=[
        pltpu.VMEM((2, ECH), jnp.int32),
        pltpu.VMEM((CH, G), jnp.int32),
        pltpu.VMEM((G,), jnp.float32),
        pltpu.VMEM((RPT,), jnp.float32),
        pltpu.VMEM_SHARED((N_PAD,), jnp.float32),
        pltpu.SemaphoreType.DMA,
    ],
)
def _sc_hist(adj_hbm, out_hbm, both, rowb, onesb, stage, deg_sh, sem):
    c = lax.axis_index("c")
    s = lax.axis_index("s")
    wid = c * NS + s

    def _z(i, _):
        stage[pl.ds(i * 16, 16)] = jnp.zeros((16,), jnp.float32)
        return 0

    lax.fori_loop(0, RPT // 16, _z, 0)
    pltpu.sync_copy(stage, deg_sh.at[pl.ds(s * RPT, RPT)])
    plsc.subcore_barrier()

    ebase = _edge_base(wid)

    def _do_groups(ng):
        def _group(g, _):
            def _mask(i, _):
                off = g * G + i * 16
                r = both[0, pl.ds(off, 16)]
                cc = both[1, pl.ds(off, 16)]
                rowb[g, pl.ds(i * 16, 16)] = r
                onesb[pl.ds(i * 16, 16)] = jnp.where(r != cc, 1.0, 0.0)
                return 0

            lax.fori_loop(0, G // 16, _mask, 0)
            pltpu.sync_copy(onesb, deg_sh.at[rowb.at[g]], add=True)
            return 0

        lax.fori_loop(0, ng, _group, 0)

    def _chunk(ch, _):
        pltpu.sync_copy(adj_hbm.at[:, pl.ds(ebase + ch * ECH, ECH)], both)
        _do_groups(CH)
        return 0

    nch = jnp.where(wid < NBIG, 10, 9)
    lax.fori_loop(0, nch, _chunk, 0)

    @pl.when(wid == NC * NS - 1)
    def _tail():
        pltpu.sync_copy(adj_hbm.at[:, pl.ds(TAILB, 512)],
                        both.at[:, pl.ds(0, 512)])
        _do_groups(4)

    plsc.subcore_barrier()
    pltpu.sync_copy(deg_sh.at[pl.ds(s * RPT, RPT)],
                    out_hbm.at[c, pl.ds(s * RPT, RPT)])


@functools.partial(
    pl.kernel,
    out_type=jax.ShapeDtypeStruct((NC, N_PAD, F), jnp.float32),
    mesh=_MESH,
    scratch_types=[
        pltpu.VMEM((2, ECH), jnp.int32),
        pltpu.VMEM((CH, G), jnp.int32),
        pltpu.VMEM((G, F), jnp.float32),
        pltpu.VMEM((G, F), jnp.float32),
        pltpu.VMEM((RCH, F), jnp.float32),
        pltpu.VMEM_SHARED((N_PAD, F), jnp.float32),
        pltpu.SemaphoreType.DMA,
        pltpu.SemaphoreType.DMA,
    ],
)
def _sc_scatter(adj_hbm, v_hbm, out_hbm, both, colb, rows0, rows1, stage,
                acc_sh, sem0, sem1):
    c = lax.axis_index("c")
    s = lax.axis_index("s")
    wid = c * NS + s

    def _z(i, _):
        for j in range(F // 16):
            stage[i, pl.ds(j * 16, 16)] = jnp.zeros((16,), jnp.float32)
        return 0

    lax.fori_loop(0, RCH, _z, 0)
    for k in range(NCH):
        pltpu.sync_copy(stage, acc_sh.at[pl.ds(s * RPT + k * RCH, RCH), :])
    plsc.subcore_barrier()

    ebase = _edge_base(wid)
    lane = lax.broadcasted_iota(jnp.int32, (16,), 0)
    bufs = (rows0, rows1)
    sems = (sem0, sem1)

    def _do_groups(ng, ch):
        def _mask(t, _):
            g = t // 8
            i = t % 8
            off = g * G + i * 16
            r = both[0, pl.ds(off, 16)]
            cc = both[1, pl.ds(off, 16)]
            trash = TRASH + ((t + ch) % 15) * 16 + lane
            colb[g, pl.ds(i * 16, 16)] = jnp.where(r == cc, trash, cc)
            return 0

        lax.fori_loop(0, ng * 8, _mask, 0)

        def _gidx(g):
            return both.at[0, pl.ds(g * G, G)]

        desc = [None] * ng
        desc[0] = pltpu.async_copy(v_hbm.at[_gidx(0)], bufs[0], sems[0])
        for g in range(ng):
            if g + 1 < ng:
                desc[g + 1] = pltpu.async_copy(
                    v_hbm.at[_gidx(g + 1)], bufs[(g + 1) % 2],
                    sems[(g + 1) % 2])
            desc[g].wait()
            pltpu.sync_copy(bufs[g % 2], acc_sh.at[colb.at[g]], add=True)

    def _chunk(ch, _):
        pltpu.sync_copy(adj_hbm.at[:, pl.ds(ebase + ch * ECH, ECH)], both)
        _do_groups(CH, ch)
        return 0

    nch = jnp.where(wid < NBIG, 10, 9)
    lax.fori_loop(0, nch, _chunk, 0)

    @pl.when(wid == NC * NS - 1)
    def _tail():
        pltpu.sync_copy(adj_hbm.at[:, pl.ds(TAILB, 512)],
                        both.at[:, pl.ds(0, 512)])
        _do_groups(4, 0)

    plsc.subcore_barrier()
    pltpu.sync_copy(acc_sh.at[pl.ds(s * RPT, RPT), :],
                    out_hbm.at[c, pl.ds(s * RPT, RPT), :])


def _dinv_sublane(deg_ref, i):
    ri = lax.broadcasted_iota(jnp.int32, (128, 128), 0)
    ci = lax.broadcasted_iota(jnp.int32, (128, 128), 1)
    pieces = []
    for j in range(BLK // 128):
        dsl = pl.ds(i * BLK + j * 128, 128)
        d = deg_ref[0:1, dsl] + deg_ref[1:2, dsl]
        dinv = jnp.where(d > 0, lax.rsqrt(d), 0.0)
        pieces.append(
            jnp.sum(jnp.where(ri == ci, dinv, 0.0), axis=1, keepdims=True)
        )
    return jnp.concatenate(pieces, axis=0)


def _tc_mm_body(x_ref, w0_ref, w1_ref, b_ref, y0_ref, y1_ref):
    xb = x_ref[...]
    y0_ref[...] = (
        jnp.dot(xb, w0_ref[...], preferred_element_type=jnp.float32)
        + b_ref[...]
    )
    y1_ref[...] = jnp.dot(xb, w1_ref[...], preferred_element_type=jnp.float32)


def _tc_vscale_body(deg_ref, y1_ref, v_ref):
    i = pl.program_id(0)
    v_ref[...] = _dinv_sublane(deg_ref, i) * y1_ref[...]


def _tc_final_body(deg_ref, y0_ref, a_ref, o_ref):
    i = pl.program_id(0)
    dinv_sub = _dinv_sublane(deg_ref, i)
    acc = a_ref[0] + a_ref[1]
    o_ref[...] = jnp.maximum(y0_ref[...] - dinv_sub * acc, 0.0)


_tc_mm = pl.pallas_call(
    _tc_mm_body,
    grid=(N_PAD // BLK,),
    in_specs=[
        pl.BlockSpec((BLK, F), lambda i: (i, 0)),
        pl.BlockSpec((F, F), lambda i: (0, 0)),
        pl.BlockSpec((F, F), lambda i: (0, 0)),
        pl.BlockSpec((1, F), lambda i: (0, 0)),
    ],
    out_specs=[
        pl.BlockSpec((BLK, F), lambda i: (i, 0)),
        pl.BlockSpec((BLK, F), lambda i: (i, 0)),
    ],
    out_shape=[
        jax.ShapeDtypeStruct((N, F), jnp.float32),
        jax.ShapeDtypeStruct((N, F), jnp.float32),
    ],
)

_tc_vscale = pl.pallas_call(
    _tc_vscale_body,
    grid=(N_PAD // BLK,),
    in_specs=[
        pl.BlockSpec((NC, N_PAD), lambda i: (0, 0)),
        pl.BlockSpec((BLK, F), lambda i: (i, 0)),
    ],
    out_specs=pl.BlockSpec((BLK, F), lambda i: (i, 0)),
    out_shape=jax.ShapeDtypeStruct((N_PAD, F), jnp.float32),
)

_tc_final = pl.pallas_call(
    _tc_final_body,
    grid=(N_PAD // BLK,),
    in_specs=[
        pl.BlockSpec((NC, N_PAD), lambda i: (0, 0)),
        pl.BlockSpec((BLK, F), lambda i: (i, 0)),
        pl.BlockSpec((NC, BLK, F), lambda i: (0, i, 0)),
    ],
    out_specs=pl.BlockSpec((BLK, F), lambda i: (i, 0)),
    out_shape=jax.ShapeDtypeStruct((N, F), jnp.float32),
)


@jax.jit
def kernel(x, adj, W0, W1, b):
    y0, y1 = _tc_mm(x, W0, W1, b.reshape(1, F))
    deg = _sc_hist(adj)
    v = _tc_vscale(deg, y1)
    acc = _sc_scatter(adj, v)
    return _tc_final(deg, y0, acc)
